# baseline (speedup 1.0000x reference)
import functools

import jax
import jax.numpy as jnp
from jax import lax
from jax.experimental import pallas as pl
from jax.experimental.pallas import tpu as pltpu
from jax.experimental.pallas import tpu_sc as plsc

_BN = 2048
_NWIN = 4


def _vq_body(zm2_ref, zz_ref, cb_ref, cc_ref, idx_ref):
    dm2 = lax.dot_general(zm2_ref[...], cb_ref[...], (((1,), (1,)), ((), ())),
                          preferred_element_type=jnp.float32)
    sq = (zz_ref[...] + cc_ref[...]) + dm2
    dist = jnp.sqrt(jnp.maximum(sq, 0.0))
    k = dist.shape[1]
    w = k // _NWIN
    iota = lax.broadcasted_iota(jnp.int32, (dist.shape[0], w), 1).astype(jnp.float32)

    def win_min(dw, base):
        m = jnp.min(dw, axis=1)
        i = jnp.min(jnp.where(dw == m[:, None], iota + base, float(k)), axis=1)
        return m, i

    cv, ci = win_min(dist[:, :w], 0)
    cv = cv.astype(jnp.bfloat16).astype(jnp.float32)
    for wi in range(1, _NWIN):
        m, i = win_min(dist[:, wi * w:(wi + 1) * w], float(wi * w))
        better = m < cv
        ci = jnp.where(better, i, ci)
        cv = jnp.where(better, m, cv).astype(jnp.bfloat16).astype(jnp.float32)
    idx_ref[...] = ci.astype(jnp.int32)


def _argmin_call(z_e, codebook):
    n, d = z_e.shape
    k = codebook.shape[0]
    zz = jnp.sum(z_e * z_e, axis=1, keepdims=True)
    cc = jnp.sum(codebook * codebook, axis=1)[None, :]
    zm2 = z_e * (-2.0)
    return pl.pallas_call(
        _vq_body,
        grid=(n // _BN,),
        in_specs=[
            pl.BlockSpec((_BN, d), lambda i: (i, 0)),
            pl.BlockSpec((_BN, 1), lambda i: (i, 0)),
            pl.BlockSpec((k, d), lambda i: (0, 0)),
            pl.BlockSpec((1, k), lambda i: (0, 0)),
        ],
        out_specs=pl.BlockSpec((_BN,), lambda i: (i,)),
        out_shape=jax.ShapeDtypeStruct((n,), jnp.int32),
    )(zm2, zz, codebook, cc)


def _sc_gather(codebook, idx):
    k, d = codebook.shape
    n = idx.shape[0]
    info = plsc.get_sparse_core_info()
    nw = info.num_cores * info.num_subcores
    b_per_w = n // nw
    nj = b_per_w // 128
    idx3 = idx.reshape(nw, nj, 128)
    pad = jnp.zeros((k, 128), jnp.float32).at[:, :d].set(codebook)
    mesh = plsc.VectorSubcoreMesh(core_axis_name="c", subcore_axis_name="s")

    @functools.partial(
        pl.kernel, mesh=mesh,
        out_type=jax.ShapeDtypeStruct((n, 128), jnp.float32),
        scratch_types=[
            pltpu.VMEM((nj, 128), jnp.int32),
            pltpu.VMEM((128, 128), jnp.float32),
            pltpu.SemaphoreType.DMA,
        ],
    )
    def gk(table_hbm, idx_hbm, out_hbm, idx_v, rows_v, sem):
        wid = lax.axis_index("s") * info.num_cores + lax.axis_index("c")
        base = wid * b_per_w
        pltpu.sync_copy(idx_hbm.at[wid], idx_v)
        for j in range(nj):
            pltpu.async_copy(table_hbm.at[idx_v.at[j]], rows_v, sem).wait()
            pltpu.sync_copy(rows_v, out_hbm.at[pl.ds(base + j * 128, 128)])

    return gk(pad, idx3)[:, :d]


def kernel(z_e, codebook):
    idx = _argmin_call(z_e, codebook)
    z_q = _sc_gather(codebook, idx)
    return (z_q, idx)


# speedup vs baseline: 1.1120x; 1.1120x over previous
import functools

import jax
import jax.numpy as jnp
from jax import lax
from jax.experimental import pallas as pl
from jax.experimental.pallas import tpu as pltpu
from jax.experimental.pallas import tpu_sc as plsc

_BN = 1024
_NWIN = 4


def _vq_body(zm2_ref, zz_ref, cb_ref, cc_ref, idx_ref):
    dm2 = lax.dot_general(zm2_ref[...], cb_ref[...], (((1,), (1,)), ((), ())),
                          preferred_element_type=jnp.float32)
    sq = (zz_ref[...] + cc_ref[...]) + dm2
    dist = jnp.sqrt(jnp.maximum(sq, 0.0))
    k = dist.shape[1]
    w = k // _NWIN
    iota = lax.broadcasted_iota(jnp.int32, (dist.shape[0], w), 1).astype(jnp.float32)

    def win_min(dw, base):
        m = jnp.min(dw, axis=1)
        i = jnp.min(jnp.where(dw == m[:, None], iota + base, float(k)), axis=1)
        return m, i

    cv, ci = win_min(dist[:, :w], 0)
    cv = cv.astype(jnp.bfloat16).astype(jnp.float32)
    for wi in range(1, _NWIN):
        m, i = win_min(dist[:, wi * w:(wi + 1) * w], float(wi * w))
        better = m < cv
        ci = jnp.where(better, i, ci)
        cv = jnp.where(better, m, cv).astype(jnp.bfloat16).astype(jnp.float32)
    idx_ref[...] = ci.astype(jnp.int32)


def _argmin_call(z_e, codebook):
    n, d = z_e.shape
    k = codebook.shape[0]
    zz = jnp.sum(z_e * z_e, axis=1, keepdims=True)
    cc = jnp.sum(codebook * codebook, axis=1)[None, :]
    zm2 = z_e * (-2.0)
    return pl.pallas_call(
        _vq_body,
        grid=(n // _BN,),
        in_specs=[
            pl.BlockSpec((_BN, d), lambda i: (i, 0)),
            pl.BlockSpec((_BN, 1), lambda i: (i, 0)),
            pl.BlockSpec((k, d), lambda i: (0, 0)),
            pl.BlockSpec((1, k), lambda i: (0, 0)),
        ],
        out_specs=pl.BlockSpec((_BN,), lambda i: (i,)),
        out_shape=jax.ShapeDtypeStruct((n,), jnp.int32),
        compiler_params=pltpu.CompilerParams(
            dimension_semantics=("parallel",)),
    )(zm2, zz, codebook, cc)


def _sc_gather(codebook, idx):
    k, d = codebook.shape
    n = idx.shape[0]
    info = plsc.get_sparse_core_info()
    nw = info.num_cores * info.num_subcores
    b_per_w = n // nw
    nj = b_per_w // 128
    idx3 = idx.reshape(nw, nj, 128)
    pad = jnp.zeros((k, 128), jnp.float32).at[:, :d].set(codebook)
    mesh = plsc.VectorSubcoreMesh(core_axis_name="c", subcore_axis_name="s")

    @functools.partial(
        pl.kernel, mesh=mesh,
        out_type=jax.ShapeDtypeStruct((n, 128), jnp.float32),
        scratch_types=[
            pltpu.VMEM((nj, 128), jnp.int32),
            pltpu.VMEM((128, 128), jnp.float32),
            pltpu.SemaphoreType.DMA,
        ],
    )
    def gk(table_hbm, idx_hbm, out_hbm, idx_v, rows_v, sem):
        wid = lax.axis_index("s") * info.num_cores + lax.axis_index("c")
        base = wid * b_per_w
        pltpu.sync_copy(idx_hbm.at[wid], idx_v)
        for j in range(nj):
            pltpu.async_copy(table_hbm.at[idx_v.at[j]], rows_v, sem).wait()
            pltpu.sync_copy(rows_v, out_hbm.at[pl.ds(base + j * 128, 128)])

    return gk(pad, idx3)[:, :d]


def kernel(z_e, codebook):
    idx = _argmin_call(z_e, codebook)
    z_q = _sc_gather(codebook, idx)
    return (z_q, idx)


# speedup vs baseline: 1.1581x; 1.0414x over previous
import functools

import jax
import jax.numpy as jnp
from jax import lax
from jax.experimental import pallas as pl
from jax.experimental.pallas import tpu as pltpu
from jax.experimental.pallas import tpu_sc as plsc

_BN = 1024
_NWIN = 4


def _vq_body(zm2_ref, zz_ref, cb_ref, cc_ref, idx_ref):
    dm2 = lax.dot_general(zm2_ref[...], cb_ref[...], (((1,), (1,)), ((), ())),
                          preferred_element_type=jnp.float32)
    sq = (zz_ref[...] + cc_ref[...]) + dm2
    dist = jnp.sqrt(sq)
    k = dist.shape[1]
    w = k // _NWIN
    iota = lax.broadcasted_iota(jnp.int32, (dist.shape[0], w), 1).astype(jnp.float32)

    def win_min(dw, base):
        m = jnp.min(dw, axis=1)
        i = jnp.min(jnp.where(dw == m[:, None], iota + base, float(k)), axis=1)
        return m, i

    cv, ci = win_min(dist[:, :w], 0)
    cv = cv.astype(jnp.bfloat16).astype(jnp.float32)
    for wi in range(1, _NWIN):
        m, i = win_min(dist[:, wi * w:(wi + 1) * w], float(wi * w))
        better = m < cv
        ci = jnp.where(better, i, ci)
        cv = jnp.where(better, m, cv).astype(jnp.bfloat16).astype(jnp.float32)
    idx_ref[...] = ci.astype(jnp.int32)


def _argmin_call(z_e, codebook):
    n, d = z_e.shape
    k = codebook.shape[0]
    zz = jnp.sum(z_e * z_e, axis=1, keepdims=True)
    cc = jnp.sum(codebook * codebook, axis=1)[None, :]
    zm2 = z_e * (-2.0)
    return pl.pallas_call(
        _vq_body,
        grid=(n // _BN,),
        in_specs=[
            pl.BlockSpec((_BN, d), lambda i: (i, 0)),
            pl.BlockSpec((_BN, 1), lambda i: (i, 0)),
            pl.BlockSpec((k, d), lambda i: (0, 0)),
            pl.BlockSpec((1, k), lambda i: (0, 0)),
        ],
        out_specs=pl.BlockSpec((_BN,), lambda i: (i,)),
        out_shape=jax.ShapeDtypeStruct((n,), jnp.int32),
        compiler_params=pltpu.CompilerParams(
            dimension_semantics=("parallel",)),
    )(zm2, zz, codebook, cc)


def _sc_gather(codebook, idx):
    k, d = codebook.shape
    n = idx.shape[0]
    info = plsc.get_sparse_core_info()
    nw = info.num_cores * info.num_subcores
    b_per_w = n // nw
    nj = b_per_w // 128
    idx3 = idx.reshape(nw, nj, 128)
    pad = jnp.zeros((k, 128), jnp.float32).at[:, :d].set(codebook)
    mesh = plsc.VectorSubcoreMesh(core_axis_name="c", subcore_axis_name="s")

    @functools.partial(
        pl.kernel, mesh=mesh,
        out_type=jax.ShapeDtypeStruct((n, 128), jnp.float32),
        scratch_types=[
            pltpu.VMEM((nj, 128), jnp.int32),
            pltpu.VMEM((128, 128), jnp.float32),
            pltpu.SemaphoreType.DMA,
        ],
    )
    def gk(table_hbm, idx_hbm, out_hbm, idx_v, rows_v, sem):
        wid = lax.axis_index("s") * info.num_cores + lax.axis_index("c")
        base = wid * b_per_w
        pltpu.sync_copy(idx_hbm.at[wid], idx_v)
        for j in range(nj):
            pltpu.async_copy(table_hbm.at[idx_v.at[j]], rows_v, sem).wait()
            pltpu.sync_copy(rows_v, out_hbm.at[pl.ds(base + j * 128, 128)])

    return gk(pad, idx3)[:, :d]


def kernel(z_e, codebook):
    idx = _argmin_call(z_e, codebook)
    z_q = _sc_gather(codebook, idx)
    return (z_q, idx)
